# Initial kernel scaffold; baseline (speedup 1.0000x reference)
import jax
import jax.numpy as jnp
from jax.experimental import pallas as pl


def kernel(input_ids, word_embeddings, position_embeddings, type_embeddings, gamma, beta):
    raise NotImplementedError("write your pallas kernel here")


# speedup vs baseline: 2.1259x; 2.1259x over previous
import functools

import jax
import jax.numpy as jnp
from jax import lax
from jax.experimental import pallas as pl
from jax.experimental.pallas import tpu as pltpu
from jax.experimental.pallas import tpu_sc as plsc

EPS = 1e-12

_NC = 2
_NS = 16
_NW = _NC * _NS
_CHUNK = 128


def _sc_gather(ids_flat, table):
    n = ids_flat.shape[0]
    d = table.shape[1]
    per_w = n // _NW
    n_chunks = per_w // _CHUNK
    mesh = plsc.VectorSubcoreMesh(core_axis_name="c", subcore_axis_name="s")

    @functools.partial(
        pl.kernel,
        mesh=mesh,
        out_type=jax.ShapeDtypeStruct((n, d), jnp.float32),
        scratch_types=[
            pltpu.VMEM((n_chunks, _CHUNK), jnp.int32),
            pltpu.VMEM((_CHUNK, d), jnp.float32),
            pltpu.SemaphoreType.DMA,
        ],
    )
    def gather_k(idx_hbm, table_hbm, out_hbm, idx_v, rows_v, sem):
        wid = lax.axis_index("s") * _NC + lax.axis_index("c")
        base = wid * per_w
        for j in range(n_chunks):
            pltpu.sync_copy(idx_hbm.at[pl.ds(base + j * _CHUNK, _CHUNK)], idx_v.at[j])
            pltpu.async_copy(table_hbm.at[idx_v.at[j]], rows_v, sem).wait()
            pltpu.sync_copy(rows_v, out_hbm.at[pl.ds(base + j * _CHUNK, _CHUNK)])

    return gather_k(ids_flat, table)


def _tc_layernorm(gathered, pos_table, typ_row, gamma2, beta2, s):
    n, d = gathered.shape
    tb = 512
    sp = s // tb

    def body(g_ref, pos_ref, typ_ref, gam_ref, bet_ref, o_ref):
        x = g_ref[...] + pos_ref[...] + typ_ref[...]
        mean = jnp.mean(x, axis=-1, keepdims=True)
        xc = x - mean
        var = jnp.mean(xc * xc, axis=-1, keepdims=True)
        o_ref[...] = (xc * lax.rsqrt(var + EPS)) * gam_ref[...] + bet_ref[...]

    return pl.pallas_call(
        body,
        grid=(n // tb,),
        in_specs=[
            pl.BlockSpec((tb, d), lambda i: (i, 0)),
            pl.BlockSpec((tb, d), lambda i: (i % sp, 0)),
            pl.BlockSpec((1, d), lambda i: (0, 0)),
            pl.BlockSpec((1, d), lambda i: (0, 0)),
            pl.BlockSpec((1, d), lambda i: (0, 0)),
        ],
        out_specs=pl.BlockSpec((tb, d), lambda i: (i, 0)),
        out_shape=jax.ShapeDtypeStruct((n, d), jnp.float32),
    )(gathered, pos_table, typ_row, gamma2, beta2)


def kernel(input_ids, word_embeddings, position_embeddings, type_embeddings, gamma, beta):
    b, s = input_ids.shape
    d = word_embeddings.shape[1]
    n = b * s
    ids_flat = input_ids.reshape(n)

    gathered = _sc_gather(ids_flat, word_embeddings)

    out = _tc_layernorm(
        gathered,
        position_embeddings,
        type_embeddings[0:1, :],
        gamma.reshape(1, d),
        beta.reshape(1, d),
        s,
    )
    return out.reshape(b, s, d)


# speedup vs baseline: 2.6886x; 1.2646x over previous
import functools

import jax
import jax.numpy as jnp
from jax import lax
from jax.experimental import pallas as pl
from jax.experimental.pallas import tpu as pltpu
from jax.experimental.pallas import tpu_sc as plsc

EPS = 1e-12

_NC = 2
_NS = 16
_NW = _NC * _NS
_CHUNK = 128


def _sc_gather(input_ids, table):
    b, s = input_ids.shape
    n = b * s
    d = table.shape[1]
    per_w = n // _NW
    segs_per_row = s // per_w
    mesh = plsc.VectorSubcoreMesh(core_axis_name="c", subcore_axis_name="s")

    @functools.partial(
        pl.kernel,
        mesh=mesh,
        out_type=jax.ShapeDtypeStruct((n, d), jnp.float32),
        scratch_types=[
            pltpu.VMEM((2, _CHUNK), jnp.int32),
            pltpu.VMEM((_CHUNK, d), jnp.float32),
            pltpu.VMEM((_CHUNK, d), jnp.float32),
            pltpu.SemaphoreType.DMA,
            pltpu.SemaphoreType.DMA,
            pltpu.SemaphoreType.DMA,
        ],
    )
    def gather_k(idx_hbm, table_hbm, out_hbm, idx_v, rows0, rows1, s0, s1, sw):
        wid = lax.axis_index("s") * _NC + lax.axis_index("c")
        row = wid // segs_per_row
        col = (wid % segs_per_row) * per_w
        base = wid * per_w
        pltpu.sync_copy(idx_hbm.at[row, pl.ds(col, _CHUNK)], idx_v.at[0])
        g0 = pltpu.async_copy(table_hbm.at[idx_v.at[0]], rows0, s0)
        pltpu.sync_copy(idx_hbm.at[row, pl.ds(col + _CHUNK, _CHUNK)], idx_v.at[1])
        g1 = pltpu.async_copy(table_hbm.at[idx_v.at[1]], rows1, s1)
        g0.wait()
        w0 = pltpu.async_copy(rows0, out_hbm.at[pl.ds(base, _CHUNK)], sw)
        g1.wait()
        w1 = pltpu.async_copy(rows1, out_hbm.at[pl.ds(base + _CHUNK, _CHUNK)], sw)
        w0.wait()
        w1.wait()

    return gather_k(input_ids, table)


def _tc_layernorm(gathered, pos_table, typ_row, gamma2, beta2, s):
    n, d = gathered.shape
    tb = s

    def body(g_ref, pos_ref, typ_ref, gam_ref, bet_ref, o_ref):
        x = g_ref[...] + pos_ref[...] + typ_ref[...]
        mean = jnp.mean(x, axis=-1, keepdims=True)
        xc = x - mean
        var = jnp.mean(xc * xc, axis=-1, keepdims=True)
        o_ref[...] = (xc * lax.rsqrt(var + EPS)) * gam_ref[...] + bet_ref[...]

    return pl.pallas_call(
        body,
        grid=(n // tb,),
        in_specs=[
            pl.BlockSpec((tb, d), lambda i: (i, 0)),
            pl.BlockSpec((tb, d), lambda i: (0, 0)),
            pl.BlockSpec((1, d), lambda i: (0, 0)),
            pl.BlockSpec((1, d), lambda i: (0, 0)),
            pl.BlockSpec((1, d), lambda i: (0, 0)),
        ],
        out_specs=pl.BlockSpec((tb, d), lambda i: (i, 0)),
        out_shape=jax.ShapeDtypeStruct((n, d), jnp.float32),
    )(gathered, pos_table, typ_row, gamma2, beta2)


def kernel(input_ids, word_embeddings, position_embeddings, type_embeddings, gamma, beta):
    b, s = input_ids.shape
    d = word_embeddings.shape[1]

    gathered = _sc_gather(input_ids, word_embeddings)

    out = _tc_layernorm(
        gathered,
        position_embeddings,
        type_embeddings[0:1, :],
        gamma.reshape(1, d),
        beta.reshape(1, d),
        s,
    )
    return out.reshape(b, s, d)


# speedup vs baseline: 2.7578x; 1.0258x over previous
import functools

import jax
import jax.numpy as jnp
from jax import lax
from jax.experimental import pallas as pl
from jax.experimental.pallas import tpu as pltpu
from jax.experimental.pallas import tpu_sc as plsc

EPS = 1e-12

_NC = 2
_NS = 16
_NW = _NC * _NS
_NCHUNK = 4


def _sc_gather(input_ids, table):
    b, s = input_ids.shape
    n = b * s
    d = table.shape[1]
    per_w = n // _NW
    ck = per_w // _NCHUNK
    segs_per_row = s // per_w
    mesh = plsc.VectorSubcoreMesh(core_axis_name="c", subcore_axis_name="s")

    @functools.partial(
        pl.kernel,
        mesh=mesh,
        out_type=jax.ShapeDtypeStruct((n, d), jnp.float32),
        scratch_types=[
            pltpu.VMEM((per_w,), jnp.int32),
            pltpu.VMEM((_NCHUNK, ck, d), jnp.float32),
            pltpu.SemaphoreType.DMA((_NCHUNK,)),
            pltpu.SemaphoreType.DMA,
        ],
    )
    def gather_k(idx_hbm, table_hbm, out_hbm, idx_v, rows_v, gsems, wsem):
        wid = lax.axis_index("s") * _NC + lax.axis_index("c")
        row = wid // segs_per_row
        col = (wid % segs_per_row) * per_w
        base = wid * per_w
        pltpu.sync_copy(idx_hbm.at[row, pl.ds(col, per_w)], idx_v)
        gathers = []
        for k in range(_NCHUNK):
            gathers.append(pltpu.async_copy(
                table_hbm.at[idx_v.at[pl.ds(k * ck, ck)]],
                rows_v.at[k], gsems.at[k]))
        for k in range(_NCHUNK):
            gathers[k].wait()
            pltpu.async_copy(rows_v.at[k], out_hbm.at[pl.ds(base + k * ck, ck)], wsem)
        for k in range(_NCHUNK):
            pltpu.make_async_copy(rows_v.at[k], out_hbm.at[pl.ds(base + k * ck, ck)], wsem).wait()

    return gather_k(input_ids, table)


def _tc_layernorm(gathered, pos_table, type_embeddings, gamma, beta, s):
    n, d = gathered.shape
    tb = 2 * s
    sp = tb // s

    def body(g_ref, pos_ref, typ_ref, gam_ref, bet_ref, o_ref):
        x = g_ref[...].reshape(sp, s, d) + pos_ref[...][None] + typ_ref[0:1, :][None]
        mean = jnp.mean(x, axis=-1, keepdims=True)
        xc = x - mean
        var = jnp.mean(xc * xc, axis=-1, keepdims=True)
        y = (xc * lax.rsqrt(var + EPS)) * gam_ref[...] + bet_ref[...]
        o_ref[...] = y.reshape(tb, d)

    typ = type_embeddings
    return pl.pallas_call(
        body,
        grid=(n // tb,),
        in_specs=[
            pl.BlockSpec((tb, d), lambda i: (i, 0)),
            pl.BlockSpec((s, d), lambda i: (0, 0)),
            pl.BlockSpec(typ.shape, lambda i: (0, 0)),
            pl.BlockSpec((d,), lambda i: (0,)),
            pl.BlockSpec((d,), lambda i: (0,)),
        ],
        out_specs=pl.BlockSpec((tb, d), lambda i: (i, 0)),
        out_shape=jax.ShapeDtypeStruct((n, d), jnp.float32),
    )(gathered, pos_table, typ, gamma, beta)


def kernel(input_ids, word_embeddings, position_embeddings, type_embeddings, gamma, beta):
    b, s = input_ids.shape
    d = word_embeddings.shape[1]

    gathered = _sc_gather(input_ids, word_embeddings)
    out = _tc_layernorm(gathered, position_embeddings, type_embeddings, gamma, beta, s)
    return out.reshape(b, s, d)
